# baseline (speedup 1.0000x reference)
import functools

import jax
import jax.numpy as jnp
from jax import lax
from jax.experimental import pallas as pl
from jax.experimental.pallas import tpu as pltpu
from jax.experimental.pallas import tpu_sc as plsc

N, K, D = 10000, 32, 128
B = N * K
NC, NS = 2, 16
NW = NC * NS
B_PER_W = B // NW
CHUNK = 200
NCHUNK = B_PER_W // CHUNK
NBUF = 2
LOOKAHEAD = 1


def _gather_sc(x, idx):
    mesh = plsc.VectorSubcoreMesh(core_axis_name="c", subcore_axis_name="s")

    @functools.partial(
        pl.kernel,
        mesh=mesh,
        out_type=jax.ShapeDtypeStruct((B, D), jnp.float32),
        scratch_types=[
            pltpu.VMEM((B_PER_W,), jnp.int32),
        ]
        + [pltpu.VMEM((CHUNK, D), jnp.float32) for _ in range(NBUF)]
        + [pltpu.SemaphoreType.DMA for _ in range(2 * NBUF)],
    )
    def k(x_hbm, idx_hbm, out_hbm, idx_v, *bufs_sems):
        bufs = bufs_sems[:NBUF]
        gsems = bufs_sems[NBUF:2 * NBUF]
        ssems = bufs_sems[2 * NBUF:]
        wid = lax.axis_index("s") * NC + lax.axis_index("c")
        base = wid * B_PER_W
        pltpu.sync_copy(idx_hbm.at[pl.ds(base, B_PER_W)], idx_v)

        def g_copy(i, b):
            off = pl.multiple_of(i * CHUNK, 8)
            return pltpu.make_async_copy(
                x_hbm.at[idx_v.at[pl.ds(off, CHUNK)]], bufs[b], gsems[b])

        def s_copy(i, b):
            off = pl.multiple_of(base + i * CHUNK, 8)
            return pltpu.make_async_copy(
                bufs[b], out_hbm.at[pl.ds(off, CHUNK)], ssems[b])

        for c in range(LOOKAHEAD):
            g_copy(c, c).start()

        def body(g, carry):
            for b in range(NBUF):
                i = g * NBUF + b
                g_copy(i, b).wait()
                s_copy(i, b).start()
                c = i + LOOKAHEAD
                bc = (b + LOOKAHEAD) % NBUF

                @pl.when(c < NCHUNK)
                def _():
                    @pl.when(c >= NBUF)
                    def _():
                        s_copy(c - NBUF, bc).wait()

                    g_copy(c, bc).start()

            return carry

        lax.fori_loop(0, NCHUNK // NBUF, body, 0)

        for b in range(NBUF):
            s_copy(NCHUNK - NBUF + b, b).wait()

    return k(x, idx)


def kernel(X, G):
    idx = G.reshape(-1).astype(jnp.int32)
    out = _gather_sc(X, idx)
    return out.reshape(N, K * D)


# speedup vs baseline: 2.2322x; 2.2322x over previous
import functools

import jax
import jax.numpy as jnp
from jax import lax
from jax.experimental import pallas as pl
from jax.experimental.pallas import tpu as pltpu
from jax.experimental.pallas import tpu_sc as plsc

N, K, D = 10000, 32, 128
B = N * K
NC, NS = 2, 16
NW = NC * NS
BLK_ROWS = 8
BLK = BLK_ROWS * K
NBLK = B // BLK
BLK_PER_W = NBLK // NW
EXTRA = NBLK % NW
NSLOT = 3


def _gather_sc(x, idx):
    mesh = plsc.VectorSubcoreMesh(core_axis_name="c", subcore_axis_name="s")

    @functools.partial(
        pl.kernel,
        mesh=mesh,
        out_type=jax.ShapeDtypeStruct((N, K * D), jnp.float32),
        scratch_types=[pltpu.VMEM((BLK,), jnp.int32) for _ in range(NSLOT)]
        + [pltpu.VMEM((BLK, D), jnp.float32) for _ in range(NSLOT)]
        + [pltpu.SemaphoreType.DMA for _ in range(2 * NSLOT)],
    )
    def k(x_hbm, idx_hbm, out_hbm, *scratch):
        ibufs = scratch[:NSLOT]
        gbufs = scratch[NSLOT:2 * NSLOT]
        isems = scratch[2 * NSLOT:3 * NSLOT]
        gsems = scratch[3 * NSLOT:]

        wid = lax.axis_index("s") * NC + lax.axis_index("c")
        b0 = wid * BLK_PER_W + jnp.minimum(wid, EXTRA)
        nblk = BLK_PER_W + jnp.where(wid < EXTRA, 1, 0)

        def i_copy(j, s):
            off = pl.multiple_of((b0 + j) * BLK, 8)
            return pltpu.make_async_copy(
                idx_hbm.at[pl.ds(off, BLK)], ibufs[s], isems[s])

        def g_copy(j, s):
            return pltpu.make_async_copy(
                x_hbm.at[ibufs[s]], gbufs[s], gsems[s])

        def store(j, s):
            off = pl.multiple_of((b0 + j) * BLK_ROWS, 8)
            pltpu.sync_copy(gbufs[s].reshape(BLK_ROWS, K * D),
                            out_hbm.at[pl.ds(off, BLK_ROWS), :])

        i_copy(0, 0).start()

        @pl.when(nblk > 1)
        def _():
            i_copy(1, 1).start()

        i_copy(0, 0).wait()
        g_copy(0, 0).start()

        def body(j, carry):
            for s in range(NSLOT):
                jj = j * NSLOT + s

                @pl.when(jj < nblk)
                def _():
                    @pl.when(jj + 2 < nblk)
                    def _():
                        i_copy(jj + 2, (s + 2) % NSLOT).start()

                    @pl.when(jj + 1 < nblk)
                    def _():
                        s1 = (s + 1) % NSLOT
                        i_copy(jj + 1, s1).wait()
                        g_copy(jj + 1, s1).start()

                    g_copy(jj, s).wait()
                    store(jj, s)

            return carry

        lax.fori_loop(0, (BLK_PER_W + 1 + NSLOT - 1) // NSLOT, body, 0)

    return k(x, idx)


def kernel(X, G):
    idx = G.reshape(-1).astype(jnp.int32)
    return _gather_sc(X, idx)
